# baseline (speedup 1.0000x reference)
import functools
import math

import jax
import jax.numpy as jnp
from jax import lax
from jax.experimental import pallas as pl
from jax.experimental.pallas import tpu as pltpu
from jax.experimental.pallas import tpu_sc as plsc

N_EXPERTS = 64
CAPACITY_FACTOR = 1.25
AUX_COEF = 0.01

_CH = 512
_SUB = 512
_NW = 32


def _tc_body(S, x_ref, wt_ref, idx_ref, gate_ref, hist_ref, aux_ref,
             cnt_acc, p_acc, z_acc):
    i = pl.program_id(0)
    E = wt_ref.shape[1]

    @pl.when(i == 0)
    def _init():
        cnt_acc[...] = jnp.zeros_like(cnt_acc)
        p_acc[...] = jnp.zeros_like(p_acc)
        z_acc[0] = jnp.float32(0.0)

    l = jnp.dot(x_ref[...], wt_ref[...], preferred_element_type=jnp.float32)
    m = jnp.max(l, axis=1, keepdims=True)
    ex = jnp.exp(l - m)
    s = jnp.sum(ex, axis=1, keepdims=True)
    idx = jnp.argmax(l, axis=1).astype(jnp.int32)
    idx_ref[0, 0, :] = idx
    gate_ref[0, 0, :] = 1.0 / s[:, 0]

    p_acc[...] += jnp.sum(ex / s, axis=0, keepdims=True)
    oh = (lax.broadcasted_iota(jnp.int32, l.shape, 1)
          == idx[:, None]).astype(jnp.float32)
    cnt_acc[...] += jnp.sum(oh, axis=0, keepdims=True)
    for j in range(_CH // _SUB):
        hist_ref[0, j, :] = jnp.sum(
            oh[j * _SUB:(j + 1) * _SUB, :], axis=0).astype(jnp.int32)

    logz = m[:, 0] + jnp.log(s[:, 0])
    z_acc[0] += jnp.sum(logz * logz)

    @pl.when(i == pl.num_programs(0) - 1)
    def _fin():
        zl = AUX_COEF * z_acc[0] / S
        lb = (AUX_COEF * E * jnp.sum(cnt_acc[...] * p_acc[...])
              / (jnp.float32(S) * jnp.float32(S)))
        aux_ref[0] = zl + lb


def _make_sc_scan(S, capacity):
    E = N_EXPERTS
    sub = S // _NW
    mesh = plsc.VectorSubcoreMesh(core_axis_name="c", subcore_axis_name="s")

    @functools.partial(
        pl.kernel,
        mesh=mesh,
        compiler_params=pltpu.CompilerParams(needs_layout_passes=False),
        out_type=jax.ShapeDtypeStruct((S,), jnp.float32),
        scratch_types=[
            pltpu.VMEM((_NW * E,), jnp.int32),
            pltpu.VMEM((sub,), jnp.int32),
            pltpu.VMEM((sub,), jnp.float32),
            pltpu.VMEM((E,), jnp.int32),
            pltpu.VMEM((sub,), jnp.float32),
        ],
    )
    def scan(e_hbm, g_hbm, h_hbm, out_hbm, all_v, e_v, g_v, cnt_v, out_v):
        wid = lax.axis_index("s") * 2 + lax.axis_index("c")
        base = wid * sub
        pltpu.sync_copy(h_hbm, all_v)
        pltpu.sync_copy(e_hbm.at[pl.ds(base, sub)], e_v)
        pltpu.sync_copy(g_hbm.at[pl.ds(base, sub)], g_v)

        def bb(t, acc):
            return tuple(a + all_v[pl.ds(t * E + 16 * j, 16)]
                         for j, a in enumerate(acc))
        z16 = jnp.zeros((16,), jnp.int32)
        acc = lax.fori_loop(0, wid, bb, (z16,) * (E // 16))
        for j in range(E // 16):
            cnt_v[pl.ds(16 * j, 16)] = acc[j]

        lane = lax.broadcasted_iota(jnp.int32, (16,), 0)
        rots = [jnp.mod(lane - k, 16) for k in range(1, 16)]

        def sb(i, carry):
            ev = e_v[pl.ds(i * 16, 16)]
            gv = g_v[pl.ds(i * 16, 16)]
            cb = plsc.load_gather(cnt_v, [ev])
            rank = jnp.zeros((16,), jnp.int32)
            tot = jnp.zeros((16,), jnp.int32)
            for k in range(1, 16):
                eq = (ev == jnp.take_along_axis(
                    ev, rots[k - 1], axis=0, mode="promise_in_bounds")
                      ).astype(jnp.int32)
                tot = tot + eq
                rank = rank + jnp.where(lane >= k, eq, 0)
            pos = cb + rank + 1
            plsc.store_scatter(cnt_v, [ev], cb + tot + 1)
            out_v[pl.ds(i * 16, 16)] = jnp.where(
                pos <= capacity, gv, jnp.float32(0.0))
            return carry
        lax.fori_loop(0, sub // 16, sb, 0)
        pltpu.sync_copy(out_v, out_hbm.at[pl.ds(base, sub)])

    return scan


def kernel(x, W):
    B, T, D = x.shape
    S = B * T
    E = W.shape[0]
    capacity = math.ceil(S / E * CAPACITY_FACTOR)
    nch = S // _CH
    nsub = S // _SUB

    x2 = x.reshape(S, D)
    wt = W.T

    idx2, gate2, hist, aux = pl.pallas_call(
        functools.partial(_tc_body, S),
        grid=(nch,),
        in_specs=[
            pl.BlockSpec((_CH, D), lambda i: (i, 0)),
            pl.BlockSpec((D, E), lambda i: (0, 0)),
        ],
        out_specs=[
            pl.BlockSpec((1, 1, _CH), lambda i: (i, 0, 0)),
            pl.BlockSpec((1, 1, _CH), lambda i: (i, 0, 0)),
            pl.BlockSpec((1, _CH // _SUB, E), lambda i: (i, 0, 0)),
            pl.BlockSpec(memory_space=pltpu.MemorySpace.SMEM),
        ],
        out_shape=[
            jax.ShapeDtypeStruct((nch, 1, _CH), jnp.int32),
            jax.ShapeDtypeStruct((nch, 1, _CH), jnp.float32),
            jax.ShapeDtypeStruct((nch, _CH // _SUB, E), jnp.int32),
            jax.ShapeDtypeStruct((1,), jnp.float32),
        ],
        scratch_shapes=[
            pltpu.VMEM((1, E), jnp.float32),
            pltpu.VMEM((1, E), jnp.float32),
            pltpu.SMEM((1,), jnp.float32),
        ],
        compiler_params=pltpu.CompilerParams(
            dimension_semantics=("arbitrary",)),
    )(x2, wt)

    e_flat = idx2.reshape(S)
    g_flat = gate2.reshape(S)
    h_flat = hist.reshape(nsub * E)
    kept = _make_sc_scan(S, capacity)(e_flat, g_flat, h_flat)
    return e_flat, kept, aux[0]


# speedup vs baseline: 1.1220x; 1.1220x over previous
import functools
import math

import jax
import jax.numpy as jnp
from jax import lax
from jax.experimental import pallas as pl
from jax.experimental.pallas import tpu as pltpu
from jax.experimental.pallas import tpu_sc as plsc

N_EXPERTS = 64
CAPACITY_FACTOR = 1.25
AUX_COEF = 0.01

_CH = 1024
_SUB = 512
_NW = 32


def _tc_body(S, x_ref, wt_ref, idx_ref, gate_ref, hist_ref, aux_ref,
             cnt_acc, p_acc, z_acc):
    i = pl.program_id(0)
    E = wt_ref.shape[1]

    @pl.when(i == 0)
    def _init():
        cnt_acc[...] = jnp.zeros_like(cnt_acc)
        p_acc[...] = jnp.zeros_like(p_acc)
        z_acc[0] = jnp.float32(0.0)

    l = jnp.dot(x_ref[...], wt_ref[...], preferred_element_type=jnp.float32)
    m = jnp.max(l, axis=1, keepdims=True)
    ex = jnp.exp(l - m)
    s = jnp.sum(ex, axis=1, keepdims=True)
    idx = jnp.argmax(l, axis=1).astype(jnp.int32)
    r = 1.0 / s
    idx_ref[0, 0, :] = idx
    gate_ref[0, 0, :] = r[:, 0]

    p_acc[...] += jnp.sum(ex * r, axis=0, keepdims=True)
    oh = (lax.broadcasted_iota(jnp.int32, l.shape, 1)
          == idx[:, None]).astype(jnp.float32)
    for j in range(_CH // _SUB):
        h = jnp.sum(oh[j * _SUB:(j + 1) * _SUB, :], axis=0, keepdims=True)
        cnt_acc[...] += h
        hist_ref[0, j, :] = h[0].astype(jnp.int32)

    logz = m[:, 0] + jnp.log(s[:, 0])
    z_acc[0] += jnp.sum(logz * logz)

    @pl.when(i == pl.num_programs(0) - 1)
    def _fin():
        zl = AUX_COEF * z_acc[0] / S
        lb = (AUX_COEF * E * jnp.sum(cnt_acc[...] * p_acc[...])
              / (jnp.float32(S) * jnp.float32(S)))
        aux_ref[0] = zl + lb


def _make_sc_scan(S, capacity):
    E = N_EXPERTS
    sub = S // _NW
    mesh = plsc.VectorSubcoreMesh(core_axis_name="c", subcore_axis_name="s")

    @functools.partial(
        pl.kernel,
        mesh=mesh,
        compiler_params=pltpu.CompilerParams(needs_layout_passes=False),
        out_type=jax.ShapeDtypeStruct((S,), jnp.float32),
        scratch_types=[
            pltpu.VMEM((_NW * E,), jnp.int32),
            pltpu.VMEM((sub,), jnp.int32),
            pltpu.VMEM((sub,), jnp.float32),
            pltpu.VMEM((E,), jnp.int32),
            pltpu.VMEM((sub,), jnp.float32),
        ],
    )
    def scan(e_hbm, g_hbm, h_hbm, out_hbm, all_v, e_v, g_v, cnt_v, out_v):
        wid = lax.axis_index("s") * 2 + lax.axis_index("c")
        base = wid * sub
        pltpu.sync_copy(h_hbm, all_v)
        pltpu.sync_copy(e_hbm.at[pl.ds(base, sub)], e_v)
        pltpu.sync_copy(g_hbm.at[pl.ds(base, sub)], g_v)

        def bb(t, acc):
            return tuple(a + all_v[pl.ds(t * E + 16 * j, 16)]
                         for j, a in enumerate(acc))
        z16 = jnp.zeros((16,), jnp.int32)
        acc = lax.fori_loop(0, wid, bb, (z16,) * (E // 16))
        for j in range(E // 16):
            cnt_v[pl.ds(16 * j, 16)] = acc[j]

        lane = lax.broadcasted_iota(jnp.int32, (16,), 0)
        rots = [jnp.mod(lane - k, 16) for k in range(1, 16)]

        def sb(i, carry):
            ev = e_v[pl.ds(i * 16, 16)]
            gv = g_v[pl.ds(i * 16, 16)]
            cb = plsc.load_gather(cnt_v, [ev])
            rank = jnp.zeros((16,), jnp.int32)
            tot = jnp.zeros((16,), jnp.int32)
            for k in range(1, 16):
                eq = (ev == jnp.take_along_axis(
                    ev, rots[k - 1], axis=0, mode="promise_in_bounds")
                      ).astype(jnp.int32)
                tot = tot + eq
                rank = rank + jnp.where(lane >= k, eq, 0)
            pos = cb + rank + 1
            plsc.store_scatter(cnt_v, [ev], cb + tot + 1)
            out_v[pl.ds(i * 16, 16)] = jnp.where(
                pos <= capacity, gv, jnp.float32(0.0))
            return carry
        lax.fori_loop(0, sub // 16, sb, 0)
        pltpu.sync_copy(out_v, out_hbm.at[pl.ds(base, sub)])

    return scan


def kernel(x, W):
    B, T, D = x.shape
    S = B * T
    E = W.shape[0]
    capacity = math.ceil(S / E * CAPACITY_FACTOR)
    nch = S // _CH
    nsub = S // _SUB

    x2 = x.reshape(S, D)
    wt = W.T

    idx2, gate2, hist, aux = pl.pallas_call(
        functools.partial(_tc_body, S),
        grid=(nch,),
        in_specs=[
            pl.BlockSpec((_CH, D), lambda i: (i, 0)),
            pl.BlockSpec((D, E), lambda i: (0, 0)),
        ],
        out_specs=[
            pl.BlockSpec((1, 1, _CH), lambda i: (i, 0, 0)),
            pl.BlockSpec((1, 1, _CH), lambda i: (i, 0, 0)),
            pl.BlockSpec((1, _CH // _SUB, E), lambda i: (i, 0, 0)),
            pl.BlockSpec(memory_space=pltpu.MemorySpace.SMEM),
        ],
        out_shape=[
            jax.ShapeDtypeStruct((nch, 1, _CH), jnp.int32),
            jax.ShapeDtypeStruct((nch, 1, _CH), jnp.float32),
            jax.ShapeDtypeStruct((nch, _CH // _SUB, E), jnp.int32),
            jax.ShapeDtypeStruct((1,), jnp.float32),
        ],
        scratch_shapes=[
            pltpu.VMEM((1, E), jnp.float32),
            pltpu.VMEM((1, E), jnp.float32),
            pltpu.SMEM((1,), jnp.float32),
        ],
        compiler_params=pltpu.CompilerParams(
            dimension_semantics=("arbitrary",)),
    )(x2, wt)

    e_flat = idx2.reshape(S)
    g_flat = gate2.reshape(S)
    h_flat = hist.reshape(nsub * E)
    kept = _make_sc_scan(S, capacity)(e_flat, g_flat, h_flat)
    return e_flat, kept, aux[0]
